# baseline (speedup 1.0000x reference)
import functools

import jax
import jax.numpy as jnp
from jax.experimental import pallas as pl

BATCH = 16384
INPUT_DIM = 768
EMBED_DIM = 256
CODEBOOK_SIZE = 1024
N_LEVELS = 4

BR = 512

def _dot(a, b, prec=jax.lax.Precision.DEFAULT):
    return jax.lax.dot(a, b, precision=prec, preferred_element_type=jnp.float32)


def _gelu_exact(v):
    return 0.5 * v * (1.0 + jax.lax.erf(v * (2.0 ** -0.5)))


def _rq_kernel(x_ref, ew1_ref, eb1_ref, ew2_ref, eb2_ref, ews_ref, ebs_ref,
               dw1_ref, db1_ref, dw2_ref, db2_ref, cb_ref,
               codes_ref, xhat_ref, recon_ref, commit_ref):
    i = pl.program_id(0)

    x = x_ref[...]
    h = _gelu_exact(_dot(x, ew1_ref[...]) + eb1_ref[...])
    z = _dot(h, ew2_ref[...]) + eb2_ref[...] + _dot(x, ews_ref[...]) + ebs_ref[...]

    HB = BR // 2
    lane_iota = jax.lax.broadcasted_iota(jnp.int32, (HB, CODEBOOK_SIZE), 1)
    rs = [z[:HB], z[HB:]]
    qsums = [jnp.zeros_like(rs[0]), jnp.zeros_like(rs[1])]
    cbs, cb2s = [], []
    for level in range(N_LEVELS):
        cb = cb_ref[level]
        cbs.append(cb)
        cb2s.append(jnp.sum(cb * cb, axis=1)[None, :])
    for level in range(N_LEVELS):
        for half in range(2):
            r = rs[half]
            cb = cbs[level]
            r2 = jnp.sum(r * r, axis=1, keepdims=True)
            dist = r2 - 2.0 * _dot(r, cb.T) + cb2s[level]
            dmin = jnp.min(dist, axis=1, keepdims=True)
            idx = jnp.min(jnp.where(dist == dmin, lane_iota, CODEBOOK_SIZE),
                          axis=1, keepdims=True)
            onehot = (idx == lane_iota).astype(jnp.float32)
            q = _dot(onehot, cb, prec=jax.lax.Precision.HIGHEST)
            rs[half] = r - q
            qsums[half] = qsums[half] + q
            codes_ref[half * HB:(half + 1) * HB, level:level + 1] = idx

    r = jnp.concatenate(rs, axis=0)
    qsum = jnp.concatenate(qsums, axis=0)
    hd = _gelu_exact(_dot(qsum, dw1_ref[...]) + db1_ref[...])
    xh = _dot(hd, dw2_ref[...]) + db2_ref[...]
    xhat_ref[...] = xh

    @pl.when(i == 0)
    def _init():
        recon_ref[...] = jnp.zeros((1, 1), jnp.float32)
        commit_ref[...] = jnp.zeros((1, 1), jnp.float32)

    recon_ref[...] += jnp.sum((xh - x) ** 2, keepdims=True)
    commit_ref[...] += jnp.sum(r * r, keepdims=True)


@jax.jit
def kernel(x, enc_w1, enc_b1, enc_w2, enc_b2, enc_ws, enc_bs,
           dec_w1, dec_b1, dec_w2, dec_b2, codebooks):
    grid = (BATCH // BR,)
    row_spec = lambda n: pl.BlockSpec((BR, n), lambda i: (i, 0))
    full2 = lambda a, b: pl.BlockSpec((a, b), lambda i: (0, 0))
    codes, x_hat, recon_sum, commit_sum = pl.pallas_call(
        _rq_kernel,
        grid=grid,
        in_specs=[
            row_spec(INPUT_DIM),
            full2(INPUT_DIM, EMBED_DIM * 2),
            full2(1, EMBED_DIM * 2),
            full2(EMBED_DIM * 2, EMBED_DIM),
            full2(1, EMBED_DIM),
            full2(INPUT_DIM, EMBED_DIM),
            full2(1, EMBED_DIM),
            full2(EMBED_DIM, EMBED_DIM * 2),
            full2(1, EMBED_DIM * 2),
            full2(EMBED_DIM * 2, INPUT_DIM),
            full2(1, INPUT_DIM),
            pl.BlockSpec((N_LEVELS, CODEBOOK_SIZE, EMBED_DIM),
                         lambda i: (0, 0, 0)),
        ],
        out_specs=[
            pl.BlockSpec((BR, N_LEVELS), lambda i: (i, 0)),
            row_spec(INPUT_DIM),
            pl.BlockSpec((1, 1), lambda i: (0, 0)),
            pl.BlockSpec((1, 1), lambda i: (0, 0)),
        ],
        out_shape=[
            jax.ShapeDtypeStruct((BATCH, N_LEVELS), jnp.int32),
            jax.ShapeDtypeStruct((BATCH, INPUT_DIM), jnp.float32),
            jax.ShapeDtypeStruct((1, 1), jnp.float32),
            jax.ShapeDtypeStruct((1, 1), jnp.float32),
        ],
    )(x, enc_w1, enc_b1.reshape(1, -1), enc_w2, enc_b2.reshape(1, -1),
      enc_ws, enc_bs.reshape(1, -1), dec_w1, dec_b1.reshape(1, -1),
      dec_w2, dec_b2.reshape(1, -1), codebooks)

    recon_loss = recon_sum[0, 0] / (BATCH * INPUT_DIM)
    commit_loss = commit_sum[0, 0] / (BATCH * EMBED_DIM)
    total_loss = recon_loss + 0.25 * commit_loss
    return (total_loss, recon_loss, commit_loss, codes, x_hat)


# speedup vs baseline: 1.3517x; 1.3517x over previous
import jax
import jax.numpy as jnp
from jax.experimental import pallas as pl
from jax.experimental.pallas import tpu as pltpu

BATCH = 16384
INPUT_DIM = 768
EMBED_DIM = 256
CODEBOOK_SIZE = 1024
N_LEVELS = 4

BR = 512

def _dot(a, b, prec=jax.lax.Precision.DEFAULT):
    return jax.lax.dot(a, b, precision=prec, preferred_element_type=jnp.float32)


def _gelu_exact(v):
    return 0.5 * v * (1.0 + jax.lax.erf(v * (2.0 ** -0.5)))


def _rq_kernel(x_ref, ew1_ref, eb1_ref, ew2_ref, eb2_ref, ews_ref, ebs_ref,
               dw1_ref, db1_ref, dw2_ref, db2_ref, cb_ref,
               codes_ref, xhat_ref, recon_ref, commit_ref, e1_ref, e2_ref):
    i = pl.program_id(0)

    @pl.when(i == 0)
    def _build_error_tables():
        row = jax.lax.broadcasted_iota(jnp.int32, (CODEBOOK_SIZE, CODEBOOK_SIZE), 0)
        col = jax.lax.broadcasted_iota(jnp.int32, (CODEBOOK_SIZE, CODEBOOK_SIZE), 1)
        eye = (row == col).astype(jnp.float32)
        for level in range(N_LEVELS):
            cb = cb_ref[level]
            e1 = cb - _dot(eye, cb)
            e2 = e1 - _dot(eye, e1)
            e1_ref[level] = e1
            e2_ref[level] = e2

    x = x_ref[...]
    h = _gelu_exact(_dot(x, ew1_ref[...]) + eb1_ref[...])
    z = _dot(h, ew2_ref[...]) + eb2_ref[...] + _dot(x, ews_ref[...]) + ebs_ref[...]

    HB = BR // 2
    lane_iota = jax.lax.broadcasted_iota(jnp.int32, (HB, CODEBOOK_SIZE), 1)
    rs = [z[:HB], z[HB:]]
    qsums = [jnp.zeros_like(rs[0]), jnp.zeros_like(rs[1])]
    cbs, cb2s = [], []
    for level in range(N_LEVELS):
        cb = cb_ref[level]
        cbs.append(cb)
        cb2s.append(jnp.sum(cb * cb, axis=1)[None, :])
    for level in range(N_LEVELS):
        for half in range(2):
            r = rs[half]
            cb = cbs[level]
            r2 = jnp.sum(r * r, axis=1, keepdims=True)
            dist = r2 - 2.0 * _dot(r, cb.T) + cb2s[level]
            dmin = jnp.min(dist, axis=1, keepdims=True)
            idx = jnp.min(jnp.where(dist == dmin, lane_iota, CODEBOOK_SIZE),
                          axis=1, keepdims=True)
            onehot = (idx == lane_iota).astype(jnp.float32)
            q = _dot(onehot, cb) + (_dot(onehot, e1_ref[level])
                                    + _dot(onehot, e2_ref[level]))
            rs[half] = r - q
            qsums[half] = qsums[half] + q
            codes_ref[half * HB:(half + 1) * HB, level:level + 1] = idx

    r = jnp.concatenate(rs, axis=0)
    qsum = jnp.concatenate(qsums, axis=0)
    hd = _gelu_exact(_dot(qsum, dw1_ref[...]) + db1_ref[...])
    xh = _dot(hd, dw2_ref[...]) + db2_ref[...]
    xhat_ref[...] = xh

    @pl.when(i == 0)
    def _init():
        recon_ref[...] = jnp.zeros((1, 1), jnp.float32)
        commit_ref[...] = jnp.zeros((1, 1), jnp.float32)

    recon_ref[...] += jnp.sum((xh - x) ** 2, keepdims=True)
    commit_ref[...] += jnp.sum(r * r, keepdims=True)


@jax.jit
def kernel(x, enc_w1, enc_b1, enc_w2, enc_b2, enc_ws, enc_bs,
           dec_w1, dec_b1, dec_w2, dec_b2, codebooks):
    grid = (BATCH // BR,)
    row_spec = lambda n: pl.BlockSpec((BR, n), lambda i: (i, 0))
    full2 = lambda a, b: pl.BlockSpec((a, b), lambda i: (0, 0))
    codes, x_hat, recon_sum, commit_sum = pl.pallas_call(
        _rq_kernel,
        grid=grid,
        in_specs=[
            row_spec(INPUT_DIM),
            full2(INPUT_DIM, EMBED_DIM * 2),
            full2(1, EMBED_DIM * 2),
            full2(EMBED_DIM * 2, EMBED_DIM),
            full2(1, EMBED_DIM),
            full2(INPUT_DIM, EMBED_DIM),
            full2(1, EMBED_DIM),
            full2(EMBED_DIM, EMBED_DIM * 2),
            full2(1, EMBED_DIM * 2),
            full2(EMBED_DIM * 2, INPUT_DIM),
            full2(1, INPUT_DIM),
            pl.BlockSpec((N_LEVELS, CODEBOOK_SIZE, EMBED_DIM),
                         lambda i: (0, 0, 0)),
        ],
        out_specs=[
            pl.BlockSpec((BR, N_LEVELS), lambda i: (i, 0)),
            row_spec(INPUT_DIM),
            pl.BlockSpec((1, 1), lambda i: (0, 0)),
            pl.BlockSpec((1, 1), lambda i: (0, 0)),
        ],
        scratch_shapes=[
            pltpu.VMEM((N_LEVELS, CODEBOOK_SIZE, EMBED_DIM), jnp.float32),
            pltpu.VMEM((N_LEVELS, CODEBOOK_SIZE, EMBED_DIM), jnp.float32),
        ],
        out_shape=[
            jax.ShapeDtypeStruct((BATCH, N_LEVELS), jnp.int32),
            jax.ShapeDtypeStruct((BATCH, INPUT_DIM), jnp.float32),
            jax.ShapeDtypeStruct((1, 1), jnp.float32),
            jax.ShapeDtypeStruct((1, 1), jnp.float32),
        ],
    )(x, enc_w1, enc_b1.reshape(1, -1), enc_w2, enc_b2.reshape(1, -1),
      enc_ws, enc_bs.reshape(1, -1), dec_w1, dec_b1.reshape(1, -1),
      dec_w2, dec_b2.reshape(1, -1), codebooks)

    recon_loss = recon_sum[0, 0] / (BATCH * INPUT_DIM)
    commit_loss = commit_sum[0, 0] / (BATCH * EMBED_DIM)
    total_loss = recon_loss + 0.25 * commit_loss
    return (total_loss, recon_loss, commit_loss, codes, x_hat)
